# Initial kernel scaffold; baseline (speedup 1.0000x reference)
import jax
import jax.numpy as jnp
from jax.experimental import pallas as pl


def kernel(x, params, t, x_initial):
    raise NotImplementedError("write your pallas kernel here")


# speedup vs baseline: 1.0553x; 1.0553x over previous
import functools

import jax
import jax.numpy as jnp
from jax import lax
from jax.experimental import pallas as pl
from jax.experimental.pallas import tpu as pltpu
from jax.experimental.pallas import tpu_sc as plsc

N = 10000
E = 320000
D = 128

NC = 2
NS = 16
NW = NC * NS

CH = 128
NCHUNK = E // CH
ITERS = (NCHUNK + NW - 1) // NW

NPAD = 10240
GCH = 128
GCHUNK = NPAD // GCH
GITER = (GCHUNK + NW - 1) // NW

NACC = 10240
RPS = NACC // NS


def _zero_vmem(buf, nrows, ncol16):
    z = jnp.zeros((16,), jnp.float32)

    def body(r, carry):
        for k in range(ncol16):
            buf[r, pl.ds(k * 16, 16)] = z
        return carry

    lax.fori_loop(0, nrows, body, 0)


def _sc_emb_cnt_body(emb_hbm, t_hbm, dst_hbm, g_hbm, cnt_hbm,
                     idx_v, rows_v, didx_v, ones_v, zc_v, acc_sh, sem):
    c = lax.axis_index("c")
    s = lax.axis_index("s")
    w = s * NC + c

    for i in range(0):
        ci = w + i * NW

        @pl.when(ci < GCHUNK)
        def _():
            off = ci * GCH
            pltpu.sync_copy(t_hbm.at[pl.ds(off, GCH)], idx_v)
            pltpu.async_copy(emb_hbm.at[idx_v], rows_v, sem).wait()
            pltpu.sync_copy(rows_v, g_hbm.at[pl.ds(off, GCH)])

    _zero_vmem(zc_v, RPS, 1)
    if False:
        pltpu.sync_copy(zc_v, acc_sh.at[s])

    e0 = jnp.where(lax.iota(jnp.int32, 16) == 0,
                   jnp.float32(1), jnp.float32(0))

    def ones_body(r, carry):
        ones_v[r, pl.ds(0, 16)] = e0
        return carry

    lax.fori_loop(0, CH, ones_body, 0)
    plsc.subcore_barrier()

    def cnt_body(i, carry):
        ci = w + i * NW

        @pl.when(ci < NCHUNK)
        def _():
            pltpu.sync_copy(dst_hbm.at[pl.ds(ci * CH, CH)], didx_v.at[0])
            pltpu.sync_copy(ones_v, acc_sh.at[didx_v.at[0]], add=True)

        return carry

    if False:
        lax.fori_loop(0, ITERS, cnt_body, 0)
    plsc.subcore_barrier()

    if False:
        pltpu.sync_copy(acc_sh.at[pl.ds(s * RPS, RPS)], zc_v)
    base = c * NACC + s * RPS
    pltpu.sync_copy(zc_v, cnt_hbm.at[pl.ds(base, RPS)])


RSPLIT = 5120
CROWS = 5248
CRPS = CROWS // NS
ITERS_N = (NCHUNK + NS - 1) // NS


def _sc_segsum_body(h_hbm, src_hbm, dst_hbm, s_hbm,
                    sidx_v, didx_v, rows_v, zb_v, acc_sh, sem):
    c = lax.axis_index("c")
    s = lax.axis_index("s")

    _zero_vmem(zb_v, CRPS, D // 16)
    pltpu.sync_copy(zb_v, acc_sh.at[pl.ds(s * CRPS, CRPS)])
    plsc.subcore_barrier()

    base_row = c * RSPLIT

    def body(i, carry):
        ci = s + i * NS

        @pl.when(ci < NCHUNK)
        def _():
            off = ci * CH
            pltpu.sync_copy(src_hbm.at[pl.ds(off, CH)], sidx_v)
            pltpu.sync_copy(dst_hbm.at[pl.ds(off, CH)], didx_v.at[0])
            pltpu.async_copy(h_hbm.at[sidx_v], rows_v, sem).wait()
            for k in range(CH // 16):
                v = didx_v[0, pl.ds(k * 16, 16)] - base_row
                ok = (v >= 0) & (v < RSPLIT)
                didx_v[0, pl.ds(k * 16, 16)] = jnp.where(
                    ok, v, jnp.int32(RSPLIT))
            pltpu.sync_copy(rows_v, acc_sh.at[didx_v.at[0]], add=True)

        return carry

    lax.fori_loop(0, ITERS_N, body, 0)
    plsc.subcore_barrier()

    pltpu.sync_copy(acc_sh.at[pl.ds(s * CRPS, CRPS)], zb_v)
    base = c * CROWS + s * CRPS
    pltpu.sync_copy(zb_v, s_hbm.at[pl.ds(base, CRPS)])


@functools.lru_cache(maxsize=None)
def _sc_kernels():
    mesh = plsc.VectorSubcoreMesh(
        core_axis_name="c", subcore_axis_name="s",
        num_cores=NC, num_subcores=NS)
    emb_cnt = pl.kernel(
        _sc_emb_cnt_body,
        out_type=[
            jax.ShapeDtypeStruct((NPAD, D), jnp.float32),
            jax.ShapeDtypeStruct((NC * NACC, 16), jnp.float32),
        ],
        mesh=mesh,
        scratch_types=[
            pltpu.VMEM((GCH,), jnp.int32),
            pltpu.VMEM((GCH, D), jnp.float32),
            pltpu.VMEM((1, CH), jnp.int32),
            pltpu.VMEM((CH, 16), jnp.float32),
            pltpu.VMEM((RPS, 16), jnp.float32),
            pltpu.VMEM_SHARED((NS, RPS, 16), jnp.float32),
            pltpu.SemaphoreType.DMA,
        ],
    )
    segsum = pl.kernel(
        _sc_segsum_body,
        out_type=jax.ShapeDtypeStruct((NC * CROWS, D), jnp.float32),
        mesh=mesh,
        scratch_types=[
            pltpu.VMEM((CH,), jnp.int32),
            pltpu.VMEM((1, CH), jnp.int32),
            pltpu.VMEM((CH, D), jnp.float32),
            pltpu.VMEM((CRPS, D), jnp.float32),
            pltpu.VMEM_SHARED((CROWS, D), jnp.float32),
            pltpu.SemaphoreType.DMA,
        ],
    )
    return emb_cnt, segsum


BR = 1000
GRID = N // BR

_row = pl.BlockSpec((BR, D), lambda i: (i, 0))
_row16 = pl.BlockSpec((BR, 16), lambda i: (i, 0))
_wmat = pl.BlockSpec((D, D), lambda i: (0, 0))
_wvec = pl.BlockSpec((1, D), lambda i: (0, 0))
_scal = pl.BlockSpec((1, 1), lambda i: (0, 0))


def _dot(a, b):
    return jnp.dot(a, b, preferred_element_type=jnp.float32)


def _tc1_body(x, g, wf_t, bf, wp_t, bp, h0_o, h1_o):
    h0 = _dot(x[...], wf_t[...]) + bf[...]
    h0_o[...] = h0
    h1_o[...] = _dot(h0, wp_t[...]) + bp[...] + g[...]


_tc1 = pl.pallas_call(
    _tc1_body,
    grid=(GRID,),
    in_specs=[_row, _row, _wmat, _wvec, _wmat, _wvec],
    out_specs=[_row, _row],
    out_shape=[jax.ShapeDtypeStruct((N, D), jnp.float32)] * 2,
)


def _tc2_body(h0, h1, s0, c0, c1, g,
              wl_t, bl, wr_t, w1_t, b1, w2_t, b2, wn_t, bn, gsel,
              h3_o, h1n_o):
    cnt = jnp.maximum(c0[:, :1] + c1[:, :1], 1.0)
    agg = s0[...] / cnt
    h2 = _dot(agg, wl_t[...]) + bl[...] + _dot(h1[...], wr_t[...])
    z = jnp.maximum(_dot(h2, w1_t[...]) + b1[...], 0.0)
    h3 = _dot(z, w2_t[...]) + b2[...] + h0[...]
    h3_o[...] = h3
    h1n_o[...] = _dot(h3, wn_t[...]) + bn[...] + gsel[0, 0] * g[...]


_tc2 = pl.pallas_call(
    _tc2_body,
    grid=(GRID,),
    in_specs=[_row, _row, _row, _row16, _row16, _row,
              _wmat, _wvec, _wmat, _wmat, _wvec, _wmat, _wvec, _wmat, _wvec,
              _scal],
    out_specs=[_row, _row],
    out_shape=[jax.ShapeDtypeStruct((N, D), jnp.float32)] * 2,
)


def kernel(x, params, t, x_initial):
    src = x_initial[0]
    dst = x_initial[1]
    tpad = jnp.pad(t.astype(jnp.int32), (0, NPAD - N))

    _sc_emb_cnt, _sc_segsum = _sc_kernels()
    g_pad, cnt = _sc_emb_cnt(params['emb'], tpad, dst)
    g = g_pad[:N]
    if True:
        cj = jax.ops.segment_sum(jnp.ones((E,), jnp.float32), dst,
                                 num_segments=N)
        c0 = jnp.tile(cj[:, None], (1, 16))
        c1 = jnp.zeros_like(c0)
    else:
        c0, c1 = cnt[:N], cnt[NACC:NACC + N]

    def wv(u):
        return u.reshape(1, D)

    p0, p1 = params['layer0'], params['layer1']

    h0, h1 = _tc1(x, g, params['W_first'].T, wv(params['b_first']),
                  p0['pre_W'].T, wv(p0['pre_b']))

    def stk(*a):
        return jnp.stack(a)

    xs = (
        stk(p0['Wl'].T, p1['Wl'].T), stk(wv(p0['bl']), wv(p1['bl'])),
        stk(p0['Wr'].T, p1['Wr'].T),
        stk(p0['post_W1'].T, p1['post_W1'].T),
        stk(wv(p0['post_b1']), wv(p1['post_b1'])),
        stk(p0['post_W2'].T, p1['post_W2'].T),
        stk(wv(p0['post_b2']), wv(p1['post_b2'])),
        stk(p1['pre_W'].T, params['W_final'].T),
        stk(wv(p1['pre_b']), wv(params['b_final'])),
        jnp.array([[[1.0]], [[0.0]]], dtype=jnp.float32),
    )

    carry = (h0, h1)
    for i in range(2):
        hres, hcur = carry
        ws = [a[i] for a in xs]
        if True:
            sfull = jax.ops.segment_sum(hcur[src], dst, num_segments=N)
        else:
            sp = _sc_segsum(hcur, src, dst)
            sfull = jnp.concatenate(
                [sp[:RSPLIT], sp[CROWS:CROWS + N - RSPLIT]], axis=0)
        carry = _tc2(hres, hcur, sfull, c0, c1, g, *ws)
    return carry[1]


# speedup vs baseline: 3.1614x; 2.9959x over previous
import functools

import jax
import jax.numpy as jnp
from jax import lax
from jax.experimental import pallas as pl
from jax.experimental.pallas import tpu as pltpu
from jax.experimental.pallas import tpu_sc as plsc

N = 10000
E = 320000
D = 128

NC = 2
NS = 16
WIN = 128

NCHUNK = E // WIN
EH = E // NC
NCH_H = EH // WIN

NPAD = 10240
GCHUNK = NPAD // WIN

NACC = 10240
CNT_RPS = NACC // NS
CNT_NZB = CNT_RPS // WIN

RSPLIT = 5120
ZWIN = 32
CROWS = 5152
SEG_NZB = CROWS // ZWIN
SEG_ZIT = (SEG_NZB + NS - 1) // NS


def _zero_rows(buf, nrows, ncols):
    @pl.loop(0, nrows)
    def _(r):
        @pl.loop(0, ncols, step=16)
        def _(k):
            buf[r, pl.ds(k, 16)] = jnp.zeros((16,), jnp.float32)


def _set_idx(idx_v, base):
    @pl.loop(0, WIN, step=16)
    def _(k):
        idx_v[0, pl.ds(k, 16)] = lax.iota(jnp.int32, 16) + base + k


def _set_idx32(idx_v, base):
    @pl.loop(0, ZWIN, step=16)
    def _(k):
        idx_v[0, pl.ds(k, 16)] = lax.iota(jnp.int32, 16) + base + k


def _sc_emb_body(emb_hbm, t_hbm, g_hbm):
    def body(i_vmem, o_vmem):
        pltpu.sync_copy(emb_hbm.at[i_vmem.at[0]], o_vmem)

    pltpu.emit_pipeline(
        body,
        grid=(GCHUNK,),
        in_specs=[pl.BlockSpec((1, WIN), index_map=lambda i: (0, i))],
        out_specs=[pl.BlockSpec((WIN, D), index_map=lambda i: (i, 0))],
        core_axis_name='s',
        dimension_semantics=(pltpu.PARALLEL,),
    )(t_hbm, g_hbm)


def _sc_segsum_body(h_hbm, s_hbm, d_hbm, o0_hbm, o1_hbm,
                    rows_v, idx_v, zidx_v, acc_sh):
    c = lax.axis_index("c")
    s = lax.axis_index("s")
    base_row = c * RSPLIT

    _zero_rows(rows_v, WIN, D)

    for i in range(SEG_ZIT):
        j = s + i * NS

        @pl.when(j < SEG_NZB)
        def _():
            _set_idx32(zidx_v, j * ZWIN)
            pltpu.sync_copy(rows_v.at[pl.ds(0, ZWIN)], acc_sh.at[zidx_v.at[0]])

    plsc.subcore_barrier()

    def abody(s_vmem, d_vmem):
        pltpu.sync_copy(h_hbm.at[s_vmem.at[0]], rows_v)

        @pl.loop(0, WIN, step=16)
        def _(k):
            v = d_vmem[0, pl.ds(k, 16)] - base_row
            ok = (v >= 0) & (v < RSPLIT)
            idx_v[0, pl.ds(k, 16)] = jnp.where(ok, v, jnp.int32(RSPLIT))

        pltpu.sync_copy(rows_v, acc_sh.at[idx_v.at[0]], add=True)

    pltpu.emit_pipeline(
        abody,
        grid=(NCHUNK,),
        in_specs=[pl.BlockSpec((1, WIN), index_map=lambda i: (0, i)),
                  pl.BlockSpec((1, WIN), index_map=lambda i: (0, i))],
        out_specs=[],
        core_axis_name='s',
        dimension_semantics=(pltpu.PARALLEL,),
    )(s_hbm, d_hbm)
    plsc.subcore_barrier()

    def readout(o_hbm):
        for i in range(SEG_ZIT):
            j = s + i * NS

            @pl.when(j < SEG_NZB)
            def _():
                base = j * ZWIN
                _set_idx32(zidx_v, base)
                pltpu.sync_copy(acc_sh.at[zidx_v.at[0]], rows_v.at[pl.ds(0, ZWIN)])
                pltpu.sync_copy(rows_v.at[pl.ds(0, ZWIN)], o_hbm.at[pl.ds(base, ZWIN)])

    @pl.when(c == 0)
    def _():
        readout(o0_hbm)

    @pl.when(c == 1)
    def _():
        readout(o1_hbm)


@functools.lru_cache(maxsize=None)
def _sc_kernels():
    mesh = plsc.VectorSubcoreMesh(
        core_axis_name="c", subcore_axis_name="s",
        num_cores=NC, num_subcores=NS)
    emb = pl.kernel(
        _sc_emb_body,
        out_type=jax.ShapeDtypeStruct((NPAD, D), jnp.float32),
        mesh=mesh,
        scratch_types=[],
    )
    segsum = pl.kernel(
        _sc_segsum_body,
        out_type=[jax.ShapeDtypeStruct((CROWS, D), jnp.float32),
                  jax.ShapeDtypeStruct((CROWS, D), jnp.float32)],
        mesh=mesh,
        scratch_types=[
            pltpu.VMEM((WIN, D), jnp.float32),
            pltpu.VMEM((1, WIN), jnp.int32),
            pltpu.VMEM((1, ZWIN), jnp.int32),
            pltpu.VMEM_SHARED((CROWS, D), jnp.float32),
        ],
    )
    return emb, segsum


BR = 1000
GRID = N // BR

_row = pl.BlockSpec((BR, D), lambda i: (i, 0))
_row16 = pl.BlockSpec((BR, 16), lambda i: (i, 0))
_wmat = pl.BlockSpec((D, D), lambda i: (0, 0))
_wvec = pl.BlockSpec((1, D), lambda i: (0, 0))
_scal = pl.BlockSpec((1, 1), lambda i: (0, 0))


def _dot(a, b):
    return jnp.dot(a, b, preferred_element_type=jnp.float32)


def _tc1_body(x, g, wf_t, bf, wp_t, bp, h0_o, h1_o):
    h0 = _dot(x[...], wf_t[...]) + bf[...]
    h0_o[...] = h0
    h1_o[...] = _dot(h0, wp_t[...]) + bp[...] + g[...]


_tc1 = pl.pallas_call(
    _tc1_body,
    grid=(GRID,),
    in_specs=[_row, _row, _wmat, _wvec, _wmat, _wvec],
    out_specs=[_row, _row],
    out_shape=[jax.ShapeDtypeStruct((N, D), jnp.float32)] * 2,
)


def _tc2_body(h0, h1, s0, c0, c1, g,
              wl_t, bl, wr_t, w1_t, b1, w2_t, b2, wn_t, bn, gsel,
              h3_o, h1n_o):
    cnt = jnp.maximum(c0[:, :1] + c1[:, :1], 1.0)
    agg = s0[...] / cnt
    h2 = _dot(agg, wl_t[...]) + bl[...] + _dot(h1[...], wr_t[...])
    z = jnp.maximum(_dot(h2, w1_t[...]) + b1[...], 0.0)
    h3 = _dot(z, w2_t[...]) + b2[...] + h0[...]
    h3_o[...] = h3
    h1n_o[...] = _dot(h3, wn_t[...]) + bn[...] + gsel[0, 0] * g[...]


_tc2 = pl.pallas_call(
    _tc2_body,
    grid=(GRID,),
    in_specs=[_row, _row, _row, _row16, _row16, _row,
              _wmat, _wvec, _wmat, _wmat, _wvec, _wmat, _wvec, _wmat, _wvec,
              _scal],
    out_specs=[_row, _row],
    out_shape=[jax.ShapeDtypeStruct((N, D), jnp.float32)] * 2,
)


def kernel(x, params, t, x_initial):
    src = x_initial[0]
    dst = x_initial[1]
    tpad = jnp.pad(t.astype(jnp.int32), (0, NPAD - N)).reshape(1, NPAD)
    src2 = src.reshape(1, E)
    dst2 = dst.reshape(1, E)

    _sc_emb, _sc_segsum = _sc_kernels()
    g = _sc_emb(params['emb'], tpad)[:N]
    ones_tab = jnp.ones((N, D), jnp.float32)
    co0, co1 = _sc_segsum(ones_tab, src2, dst2)
    c0 = jnp.concatenate([co0[:RSPLIT, :16], co1[:N - RSPLIT, :16]], axis=0)
    c1 = jnp.zeros_like(c0)

    def wv(u):
        return u.reshape(1, D)

    p0, p1 = params['layer0'], params['layer1']

    h0, h1 = _tc1(x, g, params['W_first'].T, wv(params['b_first']),
                  p0['pre_W'].T, wv(p0['pre_b']))

    xs = [
        (p0['Wl'].T, wv(p0['bl']), p0['Wr'].T,
         p0['post_W1'].T, wv(p0['post_b1']),
         p0['post_W2'].T, wv(p0['post_b2']),
         p1['pre_W'].T, wv(p1['pre_b']),
         jnp.ones((1, 1), jnp.float32)),
        (p1['Wl'].T, wv(p1['bl']), p1['Wr'].T,
         p1['post_W1'].T, wv(p1['post_b1']),
         p1['post_W2'].T, wv(p1['post_b2']),
         params['W_final'].T, wv(params['b_final']),
         jnp.zeros((1, 1), jnp.float32)),
    ]

    carry = (h0, h1)
    for i in range(2):
        hres, hcur = carry
        o0, o1 = _sc_segsum(hcur, src2, dst2)
        sfull = jnp.concatenate([o0[:RSPLIT], o1[:N - RSPLIT]], axis=0)
        carry = _tc2(hres, hcur, sfull, c0, c1, g, *xs[i])
    return carry[1]
